# Initial kernel scaffold; baseline (speedup 1.0000x reference)
import jax
import jax.numpy as jnp
from jax.experimental import pallas as pl


def kernel(x, table):
    raise NotImplementedError("write your pallas kernel here")


# speedup vs baseline: 4.6736x; 4.6736x over previous
import functools

import jax
import jax.numpy as jnp
from jax import lax
from jax.experimental import pallas as pl
from jax.experimental.pallas import tpu as pltpu
from jax.experimental.pallas import tpu_sc as plsc

_NC = 2
_NS = 16
_NW = _NC * _NS

_CHUNK = 128
_NBUF = 5


def _emb_body(n_chunks, x_hbm, table_hbm, out_hbm, idx_v, rows_v, *sems):
    D = table_hbm.shape[1]
    wid = lax.axis_index("s") * _NC + lax.axis_index("c")

    pltpu.sync_copy(x_hbm.at[wid], idx_v)

    def start_gather(j, b):
        pltpu.make_async_copy(
            table_hbm.at[idx_v.at[j]], rows_v.at[b], sems[b]
        ).start()

    def wait_gather(j, b):
        pltpu.make_async_copy(
            table_hbm.at[idx_v.at[j]], rows_v.at[b], sems[b]
        ).wait()

    for b in range(_NBUF):
        start_gather(b, b)

    @pl.loop(0, n_chunks // _NBUF)
    def _outer(o):
        for b in range(_NBUF):
            j = o * _NBUF + b
            wait_gather(j, b)
            pltpu.sync_copy(rows_v.at[b], out_hbm.at[wid, j])
            nxt = j + _NBUF

            @pl.when(nxt < n_chunks)
            def _():
                start_gather(nxt, b)


@jax.jit
def kernel(x, table):
    B0, B1 = x.shape
    V, D = table.shape
    B = B0 * B1
    assert B % (_NW * _CHUNK) == 0
    b_per_w = B // _NW
    n_chunks = b_per_w // _CHUNK
    assert n_chunks % _NBUF == 0

    idx = x.reshape(_NW, n_chunks, _CHUNK).astype(jnp.int32)

    mesh = plsc.VectorSubcoreMesh(core_axis_name="c", subcore_axis_name="s")
    run = pl.kernel(
        functools.partial(_emb_body, n_chunks),
        out_type=jax.ShapeDtypeStruct((_NW, n_chunks, _CHUNK, D), jnp.float32),
        mesh=mesh,
        scratch_types=[
            pltpu.VMEM((n_chunks, _CHUNK), jnp.int32),
            pltpu.VMEM((_NBUF, _CHUNK, D), jnp.float32),
        ]
        + [pltpu.SemaphoreType.DMA] * _NBUF,
        compiler_params=pltpu.CompilerParams(use_tc_tiling_on_sc=False),
        name="sc_embedding_gather",
    )
    out = run(idx, table)
    return out.reshape(B0, B1, D)


# speedup vs baseline: 4.6932x; 1.0042x over previous
import functools

import jax
import jax.numpy as jnp
from jax import lax
from jax.experimental import pallas as pl
from jax.experimental.pallas import tpu as pltpu
from jax.experimental.pallas import tpu_sc as plsc

_NC = 2
_NS = 16
_NW = _NC * _NS

_CHUNK = 256
_NBUF = 5


def _emb_body(n_chunks, x_hbm, table_hbm, out_hbm, idx_v, rows_v, *sems):
    D = table_hbm.shape[1]
    wid = lax.axis_index("s") * _NC + lax.axis_index("c")

    pltpu.sync_copy(x_hbm.at[wid], idx_v)

    def start_gather(j, b):
        pltpu.make_async_copy(
            table_hbm.at[idx_v.at[j]], rows_v.at[b], sems[b]
        ).start()

    def wait_gather(j, b):
        pltpu.make_async_copy(
            table_hbm.at[idx_v.at[j]], rows_v.at[b], sems[b]
        ).wait()

    for b in range(_NBUF):
        start_gather(b, b)

    @pl.loop(0, n_chunks // _NBUF)
    def _outer(o):
        for b in range(_NBUF):
            j = o * _NBUF + b
            wait_gather(j, b)
            pltpu.sync_copy(rows_v.at[b], out_hbm.at[wid, j])
            nxt = j + _NBUF

            @pl.when(nxt < n_chunks)
            def _():
                start_gather(nxt, b)


@jax.jit
def kernel(x, table):
    B0, B1 = x.shape
    V, D = table.shape
    B = B0 * B1
    assert B % (_NW * _CHUNK) == 0
    b_per_w = B // _NW
    n_chunks = b_per_w // _CHUNK
    assert n_chunks % _NBUF == 0

    idx = x.reshape(_NW, n_chunks, _CHUNK).astype(jnp.int32)

    mesh = plsc.VectorSubcoreMesh(core_axis_name="c", subcore_axis_name="s")
    run = pl.kernel(
        functools.partial(_emb_body, n_chunks),
        out_type=jax.ShapeDtypeStruct((_NW, n_chunks, _CHUNK, D), jnp.float32),
        mesh=mesh,
        scratch_types=[
            pltpu.VMEM((n_chunks, _CHUNK), jnp.int32),
            pltpu.VMEM((_NBUF, _CHUNK, D), jnp.float32),
        ]
        + [pltpu.SemaphoreType.DMA] * _NBUF,
        compiler_params=pltpu.CompilerParams(use_tc_tiling_on_sc=False),
        name="sc_embedding_gather",
    )
    out = run(idx, table)
    return out.reshape(B0, B1, D)
